# baseline (speedup 1.0000x reference)
import jax
import jax.numpy as jnp
from jax import lax
from jax.experimental import pallas as pl
from jax.experimental.pallas import tpu as pltpu
from jax.experimental.pallas import tpu_sc as plsc

_ROWS = 8192
_COLS = 2048
_NC = 2
_NS = 16
_NW = _NC * _NS
_RPW = _ROWS // _NW
_CROWS = 16
_NB = 3
_NCH = _RPW // _CROWS


def _tec_body(src_hbm, dst_hbm, buf, *sems):
    sin = sems[:_NB]
    sout = sems[_NB:]
    wid = lax.axis_index("s") * _NC + lax.axis_index("c")
    sid = lax.axis_index("s")
    base = wid * _RPW

    def in_copy(j):
        return pltpu.make_async_copy(
            src_hbm.at[pl.ds(base + j * _CROWS, _CROWS), :],
            buf.at[sid, j % _NB], sin[j % _NB])

    def out_copy(j):
        return pltpu.make_async_copy(
            buf.at[sid, j % _NB],
            dst_hbm.at[pl.ds(base + j * _CROWS, _CROWS), :], sout[j % _NB])

    for b in range(_NB):
        in_copy(b).start()
    for j in range(_NCH):
        if j >= _NB:
            out_copy(j - _NB).wait()
            in_copy(j).start()
        in_copy(j).wait()
        out_copy(j).start()
    for j in range(_NCH - _NB, _NCH):
        out_copy(j).wait()


def kernel(inputs, pos_table):
    del inputs
    k = pl.kernel(
        _tec_body,
        out_type=jax.ShapeDtypeStruct((_ROWS, _COLS), jnp.float32),
        mesh=plsc.VectorSubcoreMesh(core_axis_name="c", subcore_axis_name="s"),
        scratch_types=(
            [pltpu.VMEM_SHARED((_NS, _NB, _CROWS, _COLS), jnp.float32)]
            + [pltpu.SemaphoreType.DMA] * (2 * _NB)
        ),
    )
    return k(pos_table)
